# baseline (speedup 1.0000x reference)
import functools
import jax
import jax.numpy as jnp
from jax import lax
from jax.experimental import pallas as pl
from jax.experimental.pallas import tpu as pltpu
from jax.experimental.pallas import tpu_sc as plsc

D = 64
DP = D // 2
S = 200
NG = 13
CHUNK = 128
K = 8
PF = 5
RING = K * CHUNK
BIAS_PAD = 1000064


@jax.jit
def _run(sl3, embed, wtab, bias):
    B = embed.shape[0]
    info = plsc.get_sparse_core_info()
    NC, NS = info.num_cores, info.num_subcores
    NW = NC * NS
    spw = B // NW
    nstream = spw * S // CHUNK
    mesh = plsc.VectorSubcoreMesh(core_axis_name="c", subcore_axis_name="s")

    @functools.partial(
        pl.kernel,
        out_type=jax.ShapeDtypeStruct((2 * B, 128), jnp.float32),
        mesh=mesh,
        compiler_params=pltpu.CompilerParams(
            needs_layout_passes=False, use_tc_tiling_on_sc=False),
        scratch_types=[
            pltpu.VMEM((nstream, CHUNK), jnp.int32),
            pltpu.VMEM((spw, D), jnp.float32),
            pltpu.VMEM((RING, D), jnp.float32),
            pltpu.VMEM((RING,), jnp.float32),
            pltpu.VMEM((4, 256), jnp.float32),
            pltpu.SemaphoreType.DMA,
            pltpu.SemaphoreType.DMA,
            pltpu.SemaphoreType.DMA,
        ],
    )
    def k(sl_hbm, embed_hbm, w_hbm, bias_hbm, out_hbm,
          idx_all, emb_all, rows_ring, bias_ring, out_stage,
          sem_r, sem_b, sem_o):
        cid = lax.axis_index("c")
        sid = lax.axis_index("s")
        wid = sid * NC + cid
        base = wid * spw
        iota = lax.iota(jnp.int32, 16)
        zvec = jnp.zeros((16,), jnp.int32)

        pltpu.sync_copy(sl_hbm.at[pl.ds(wid * nstream, nstream)], idx_all)
        pltpu.sync_copy(embed_hbm.at[pl.ds(base, spw)], emb_all)

        def issue(j):
            slot = j % K
            pltpu.make_async_copy(
                w_hbm.at[idx_all.at[j]],
                rows_ring.at[pl.ds(slot * CHUNK, CHUNK)], sem_r).start()
            pltpu.make_async_copy(
                bias_hbm.at[idx_all.at[j]],
                bias_ring.at[pl.ds(slot * CHUNK, CHUNK)], sem_b).start()

        def wait_chunk():
            pltpu.make_async_copy(
                w_hbm.at[pl.ds(0, CHUNK)],
                rows_ring.at[pl.ds(0, CHUNK)], sem_r).wait()
            pltpu.make_async_copy(
                bias_hbm.at[pl.ds(0, CHUNK)],
                bias_ring.at[pl.ds(0, CHUNK)], sem_b).wait()

        def wait_out_half(buf):
            pltpu.make_async_copy(
                out_stage.at[buf, pl.ds(0, 128)],
                out_hbm.at[0], sem_o).wait()

        def prologue(j, carry):
            issue(j)
            return carry

        lax.fori_loop(0, PF, prologue, 0)

        def sample_body(i, carry):
            need_prev = (25 * i + 15) >> 4
            need_cur = (25 * i + 40) >> 4
            j1 = need_prev + PF
            j2 = j1 + 1
            jtgt = jnp.minimum(need_cur + PF, nstream)

            @pl.when(j1 < jtgt)
            def _():
                issue(j1)

            @pl.when(j2 < jtgt)
            def _():
                issue(j2)

            wait_chunk()

            @pl.when(need_cur - need_prev > 1)
            def _():
                wait_chunk()

            @pl.when(i >= 4)
            def _():
                buf_old = (i - 4) % 4
                wait_out_half(buf_old)
                wait_out_half(buf_old)

            buf = i % 4
            fvecs = [((200 * i + 16 * g) + iota) % RING for g in range(NG)]
            accs0 = tuple(
                plsc.load_gather(bias_ring, [fvecs[g]]) for g in range(NG))
            isplat = zvec + i

            def dbody(dp, accs):
                dcol = (dp + iota) & (D - 1)
                e0 = plsc.load_gather(emb_all, [isplat, dcol])
                return tuple(
                    a + plsc.load_gather(rows_ring, [fvecs[g], dcol]) * e0
                    for g, a in enumerate(accs))

            accs = lax.fori_loop(0, D, dbody, accs0)
            st = out_stage.at[buf]
            for g in range(NG):
                st[pl.ds(16 * g, 16)] = accs[g]
            pltpu.make_async_copy(
                out_stage.at[buf, pl.ds(0, 128)],
                out_hbm.at[2 * (base + i)], sem_o).start()
            pltpu.make_async_copy(
                out_stage.at[buf, pl.ds(128, 128)],
                out_hbm.at[2 * (base + i) + 1], sem_o).start()
            return carry

        lax.fori_loop(0, spw, sample_body, 0)

        def out_drain(j, carry):
            wait_out_half(0)
            return carry

        lax.fori_loop(0, 8, out_drain, 0)

    return k(sl3, embed, wtab, bias)


def kernel(embed, shortlist, W, b):
    B = embed.shape[0]
    rows = W.shape[0]
    sl3 = shortlist.astype(jnp.int32).reshape(B * S // CHUNK, CHUNK)
    wtab = W + embed[0, 0] * jnp.float32(0.0)
    bias = jnp.pad(b.reshape(-1), (0, BIAS_PAD - rows))
    out2 = _run(sl3, embed, wtab, bias)
    return out2.reshape(B, 256)[:, :S]


# speedup vs baseline: 1.5700x; 1.5700x over previous
import functools
import jax
import jax.numpy as jnp
from jax import lax
from jax.experimental import pallas as pl
from jax.experimental.pallas import tpu as pltpu
from jax.experimental.pallas import tpu_sc as plsc

D = 64
DP = D // 2
S = 200
NG = 13
CHUNK = 128
K = 8
PF = 5
RING = K * CHUNK
BIAS_PAD = 1000064


@jax.jit
def _run(sl3, embed, wtab, bias):
    B = embed.shape[0]
    info = plsc.get_sparse_core_info()
    NC, NS = info.num_cores, info.num_subcores
    NW = NC * NS
    spw = B // NW
    nstream = spw * S // CHUNK
    mesh = plsc.VectorSubcoreMesh(core_axis_name="c", subcore_axis_name="s")

    @functools.partial(
        pl.kernel,
        out_type=jax.ShapeDtypeStruct((2 * B, 128), jnp.float32),
        mesh=mesh,
        compiler_params=pltpu.CompilerParams(
            needs_layout_passes=False, use_tc_tiling_on_sc=False),
        scratch_types=[
            pltpu.VMEM((nstream, CHUNK), jnp.int32),
            pltpu.VMEM((spw, D), jnp.float32),
            pltpu.VMEM((RING, D), jnp.float32),
            pltpu.VMEM((RING,), jnp.float32),
            pltpu.VMEM((4, 256), jnp.float32),
            pltpu.SemaphoreType.DMA,
            pltpu.SemaphoreType.DMA,
            pltpu.SemaphoreType.DMA,
        ],
    )
    def k(sl_hbm, embed_hbm, w_hbm, bias_hbm, out_hbm,
          idx_all, emb_all, rows_ring, bias_ring, out_stage,
          sem_r, sem_b, sem_o):
        cid = lax.axis_index("c")
        sid = lax.axis_index("s")
        wid = sid * NC + cid
        base = wid * spw
        iota = lax.iota(jnp.int32, 16)
        zvec = jnp.zeros((16,), jnp.int32)

        pltpu.sync_copy(sl_hbm.at[pl.ds(wid * nstream, nstream)], idx_all)
        pltpu.sync_copy(embed_hbm.at[pl.ds(base, spw)], emb_all)

        def issue(j):
            slot = j % K
            pltpu.make_async_copy(
                w_hbm.at[idx_all.at[j]],
                rows_ring.at[pl.ds(slot * CHUNK, CHUNK)], sem_r).start()
            pltpu.make_async_copy(
                bias_hbm.at[idx_all.at[j]],
                bias_ring.at[pl.ds(slot * CHUNK, CHUNK)], sem_b).start()

        def wait_chunk():
            pltpu.make_async_copy(
                w_hbm.at[pl.ds(0, CHUNK)],
                rows_ring.at[pl.ds(0, CHUNK)], sem_r).wait()
            pltpu.make_async_copy(
                bias_hbm.at[pl.ds(0, CHUNK)],
                bias_ring.at[pl.ds(0, CHUNK)], sem_b).wait()

        def wait_out_half(buf):
            pltpu.make_async_copy(
                out_stage.at[buf, pl.ds(0, 128)],
                out_hbm.at[0], sem_o).wait()

        def prologue(j, carry):
            issue(j)
            return carry

        lax.fori_loop(0, PF, prologue, 0)

        def sample_body(i, carry):
            need_prev = (25 * i + 15) >> 4
            need_cur = (25 * i + 40) >> 4
            j1 = need_prev + PF
            j2 = j1 + 1
            jtgt = jnp.minimum(need_cur + PF, nstream)

            @pl.when(j1 < jtgt)
            def _():
                issue(j1)

            @pl.when(j2 < jtgt)
            def _():
                issue(j2)

            wait_chunk()

            @pl.when(need_cur - need_prev > 1)
            def _():
                wait_chunk()

            @pl.when(i >= 4)
            def _():
                buf_old = (i - 4) % 4
                wait_out_half(buf_old)
                wait_out_half(buf_old)

            buf = i % 4
            fvecs = [((200 * i + 16 * g) + iota) % RING for g in range(NG)]
            accs0 = tuple(
                plsc.load_gather(bias_ring, [fvecs[g]]) for g in range(NG))
            isplat = zvec + i

            def dbody(dp, accs):
                dcol = (dp + iota) & (D - 1)
                e0 = plsc.load_gather(emb_all, [isplat, dcol])
                return tuple(
                    a + plsc.load_gather(rows_ring, [fvecs[g], dcol]) * e0
                    for g, a in enumerate(accs))

            accs = lax.fori_loop(0, D, dbody, accs0)
            st = out_stage.at[buf]
            for g in range(NG):
                st[pl.ds(16 * g, 16)] = accs[g]
            pltpu.make_async_copy(
                out_stage.at[buf, pl.ds(0, 128)],
                out_hbm.at[2 * (base + i)], sem_o).start()
            pltpu.make_async_copy(
                out_stage.at[buf, pl.ds(128, 128)],
                out_hbm.at[2 * (base + i) + 1], sem_o).start()
            return carry

        lax.fori_loop(0, spw, sample_body, 0)

        def out_drain(j, carry):
            wait_out_half(0)
            return carry

        lax.fori_loop(0, 8, out_drain, 0)

    return k(sl3, embed, wtab, bias)


def kernel(embed, shortlist, W, b):
    B = embed.shape[0]
    rows = W.shape[0]
    sl3 = (shortlist.astype(jnp.int32) * 2).reshape(B * S // CHUNK, CHUNK)
    wtab = jnp.pad(W, ((0, 7), (0, 64))).reshape(2 * (rows + 7), D)
    bias = jnp.pad(b.reshape(-1), (0, BIAS_PAD - rows))
    out2 = _run(sl3, embed, wtab, bias)
    return out2.reshape(B, 256)[:, :S]
